# baseline (speedup 1.0000x reference)
import functools

import jax
import jax.numpy as jnp
from jax import lax
from jax.experimental import pallas as pl
from jax.experimental.pallas import tpu as pltpu
from jax.experimental.pallas import tpu_sc as plsc

VOCAB = 1000000
B = 16384
D0 = 64
D1 = 16

_info = plsc.get_sparse_core_info()
_NC = _info.num_cores
_NS = _info.num_subcores
_NW = _NC * _NS
_BPW = B // _NW


def _gather_body(idx_hbm, np0_hbm, np1_hbm, gp0_hbm, gp1_hbm,
                 o_np0, o_np1, o_gp0, o_gp1,
                 idx_v, r_np0, r_np1, r_gp0, r_gp1, sem_in, sem_out):
    wid = lax.axis_index("s") * _NC + lax.axis_index("c")
    base = wid * _BPW
    pltpu.sync_copy(idx_hbm.at[pl.ds(base, _BPW)], idx_v)
    gathers = [
        pltpu.async_copy(np0_hbm.at[idx_v], r_np0, sem_in),
        pltpu.async_copy(np1_hbm.at[idx_v], r_np1, sem_in),
        pltpu.async_copy(gp0_hbm.at[idx_v], r_gp0, sem_in),
        pltpu.async_copy(gp1_hbm.at[idx_v], r_gp1, sem_in),
    ]
    for c in gathers:
        c.wait()
    stores = [
        pltpu.async_copy(r_np0, o_np0.at[pl.ds(base, _BPW)], sem_out),
        pltpu.async_copy(r_np1, o_np1.at[pl.ds(base, _BPW)], sem_out),
        pltpu.async_copy(r_gp0, o_gp0.at[pl.ds(base, _BPW)], sem_out),
        pltpu.async_copy(r_gp1, o_gp1.at[pl.ds(base, _BPW)], sem_out),
    ]
    for c in stores:
        c.wait()


@jax.jit
def _gather_all(idx, np0, np1, gp0, gp1):
    mesh = plsc.VectorSubcoreMesh(core_axis_name="c", subcore_axis_name="s")
    f = functools.partial(
        pl.kernel,
        mesh=mesh,
        out_type=[
            jax.ShapeDtypeStruct((B, D0), jnp.float32),
            jax.ShapeDtypeStruct((B, D1), jnp.float32),
            jax.ShapeDtypeStruct((B, D0), jnp.float32),
            jax.ShapeDtypeStruct((B, D1), jnp.float32),
        ],
        scratch_types=[
            pltpu.VMEM((_BPW,), jnp.int32),
            pltpu.VMEM((_BPW, D0), jnp.float32),
            pltpu.VMEM((_BPW, D1), jnp.float32),
            pltpu.VMEM((_BPW, D0), jnp.float32),
            pltpu.VMEM((_BPW, D1), jnp.float32),
            pltpu.SemaphoreType.DMA,
            pltpu.SemaphoreType.DMA,
        ],
        compiler_params=pltpu.CompilerParams(use_tc_tiling_on_sc=False),
    )(_gather_body)
    return f(idx, np0, np1, gp0, gp1)


def kernel(idx, np0, np1, gp0, gp1):
    o0, o1, o2, o3 = _gather_all(idx, np0, np1, gp0, gp1)
    return ([o0, o1], [o2, o3])


# speedup vs baseline: 8.1408x; 8.1408x over previous
import functools

import jax
import jax.numpy as jnp
from jax import lax
from jax.experimental import pallas as pl
from jax.experimental.pallas import tpu as pltpu
from jax.experimental.pallas import tpu_sc as plsc

VOCAB = 1000000
B = 16384
D0 = 64
D1 = 16

_info = plsc.get_sparse_core_info()
_NC = _info.num_cores
_NS = _info.num_subcores
_NW = _NC * _NS
_BPW = B // _NW
_C = 8
_G = 16


def _gather_body(idx_hbm, t0, t1, t2, t3, o0, o1, o2, o3,
                 idx_sh, idx_s, idx_vv, ga0, ga1, ga2, ga3,
                 gb0, gb1, gb2, gb3, b0, b1, b2, b3,
                 sem_a, sem_b, sem_s):
    wid = lax.axis_index("s") * _NC + lax.axis_index("c")
    base = wid * _BPW
    pltpu.sync_copy(idx_hbm.at[pl.ds(base, _BPW)], idx_vv)
    pltpu.sync_copy(idx_hbm.at[pl.ds(base, _BPW)], idx_sh.at[wid])
    pltpu.sync_copy(idx_sh.at[wid], idx_s)

    i16 = lax.iota(jnp.int32, 16)
    d1v = lax.rem(i16, 8)
    bufs_a = ((ga0, b0, 8), (ga1, b1, 2), (ga2, b2, 8), (ga3, b3, 2))
    bufs_b = ((gb0, b0, 8), (gb1, b1, 2), (gb2, b2, 8), (gb3, b3, 2))

    def issue_chunk(ck, gbufs, sem):
        def issue(jj, c2):
            v = idx_s[ck * _C + jj]
            v16 = pl.multiple_of((v >> 4) << 4, _G)
            s = jj * _G
            for t, (g, _, _) in zip((t0, t1, t2, t3), gbufs):
                pltpu.make_async_copy(
                    t.at[:, :, pl.ds(v16, _G)], g.at[:, :, pl.ds(s, _G)],
                    sem).start()
            return c2

        lax.fori_loop(0, _C, issue, 0)

    def wait_chunk(gbufs, sem):
        for t, (g, _, _) in zip((t0, t1, t2, t3), gbufs):
            pltpu.make_async_copy(t.at[:, :, pl.ds(0, _C * _G)], g,
                                  sem).wait()

    def extract_chunk(ck, gbufs):
        def extract(jj, c2):
            jcol = ck * _C + jj
            v_vec = plsc.load_gather(idx_vv, [i16 * 0 + jcol])
            lane = lax.rem(v_vec, _G) + jj * _G
            for gbuf, obuf, nfb in gbufs:
                for g in range(nfb * 8 // 16):
                    d0 = 2 * g + i16 // 8
                    vals = plsc.load_gather(gbuf, [d0, d1v, lane])
                    plsc.store_scatter(obuf, [d0, d1v, i16 * 0 + jcol], vals)
            return c2

        lax.fori_loop(0, _C, extract, 0)

    n_chunks = _BPW // _C

    issue_chunk(0, bufs_a, sem_a)

    def step(k, carry):
        issue_chunk(2 * k + 1, bufs_b, sem_b)
        wait_chunk(bufs_a, sem_a)
        extract_chunk(2 * k, bufs_a)

        @pl.when(2 * k + 2 < n_chunks)
        def _():
            issue_chunk(2 * k + 2, bufs_a, sem_a)

        wait_chunk(bufs_b, sem_b)
        extract_chunk(2 * k + 1, bufs_b)
        return carry

    lax.fori_loop(0, n_chunks // 2, step, 0)

    stores = [
        pltpu.async_copy(b0, o0.at[:, :, pl.ds(base, _BPW)], sem_s),
        pltpu.async_copy(b1, o1.at[:, :, pl.ds(base, _BPW)], sem_s),
        pltpu.async_copy(b2, o2.at[:, :, pl.ds(base, _BPW)], sem_s),
        pltpu.async_copy(b3, o3.at[:, :, pl.ds(base, _BPW)], sem_s),
    ]
    for c in stores:
        c.wait()


@jax.jit
def _gather_all(idx, np0, np1, gp0, gp1):
    t0 = np0.T.reshape(D0 // 8, 8, VOCAB)
    t1 = np1.T.reshape(D1 // 8, 8, VOCAB)
    t2 = gp0.T.reshape(D0 // 8, 8, VOCAB)
    t3 = gp1.T.reshape(D1 // 8, 8, VOCAB)
    mesh = plsc.VectorSubcoreMesh(core_axis_name="c", subcore_axis_name="s")
    f = functools.partial(
        pl.kernel,
        mesh=mesh,
        out_type=[
            jax.ShapeDtypeStruct((D0 // 8, 8, B), jnp.float32),
            jax.ShapeDtypeStruct((D1 // 8, 8, B), jnp.float32),
            jax.ShapeDtypeStruct((D0 // 8, 8, B), jnp.float32),
            jax.ShapeDtypeStruct((D1 // 8, 8, B), jnp.float32),
        ],
        scratch_types=[
            pltpu.VMEM_SHARED((_NW, _BPW), jnp.int32),
            pltpu.SMEM((_BPW,), jnp.int32),
            pltpu.VMEM((_BPW,), jnp.int32),
            pltpu.VMEM((D0 // 8, 8, _C * _G), jnp.float32),
            pltpu.VMEM((D1 // 8, 8, _C * _G), jnp.float32),
            pltpu.VMEM((D0 // 8, 8, _C * _G), jnp.float32),
            pltpu.VMEM((D1 // 8, 8, _C * _G), jnp.float32),
            pltpu.VMEM((D0 // 8, 8, _C * _G), jnp.float32),
            pltpu.VMEM((D1 // 8, 8, _C * _G), jnp.float32),
            pltpu.VMEM((D0 // 8, 8, _C * _G), jnp.float32),
            pltpu.VMEM((D1 // 8, 8, _C * _G), jnp.float32),
            pltpu.VMEM((D0 // 8, 8, _BPW), jnp.float32),
            pltpu.VMEM((D1 // 8, 8, _BPW), jnp.float32),
            pltpu.VMEM((D0 // 8, 8, _BPW), jnp.float32),
            pltpu.VMEM((D1 // 8, 8, _BPW), jnp.float32),
            pltpu.SemaphoreType.DMA,
            pltpu.SemaphoreType.DMA,
            pltpu.SemaphoreType.DMA,
        ],
        compiler_params=pltpu.CompilerParams(needs_layout_passes=False),
    )(_gather_body)
    q0, q1, q2, q3 = f(idx, t0, t1, t2, t3)
    return (q0.reshape(D0, B).T, q1.reshape(D1, B).T,
            q2.reshape(D0, B).T, q3.reshape(D1, B).T)


def kernel(idx, np0, np1, gp0, gp1):
    o0, o1, o2, o3 = _gather_all(idx, np0, np1, gp0, gp1)
    return ([o0, o1], [o2, o3])
